# Initial kernel scaffold; baseline (speedup 1.0000x reference)
import jax
import jax.numpy as jnp
from jax.experimental import pallas as pl


def kernel(x, edge_index, Wl1, Wr1, b1, Wl2, Wr2, b2, Wl3, Wr3, b3):
    raise NotImplementedError("write your pallas kernel here")


# speedup vs baseline: 8.0413x; 8.0413x over previous
import functools

import jax
import jax.numpy as jnp
from jax import lax
from jax.experimental import pallas as pl
from jax.experimental.pallas import tpu as pltpu
from jax.experimental.pallas import tpu_sc as plsc

N, E, D = 10000, 320000, 128
NC, NS = 2, 16
NW = NC * NS
EPT = E // NW
C = 125
NCHUNK = EPT // C
NPAD = 10240
RPT = NPAD // NS
ZB = 16
ZR = 128
NG = 20
NGRP = NCHUNK // NG


def _make_sc_agg(W):
    mesh = plsc.VectorSubcoreMesh(core_axis_name="c", subcore_axis_name="s")

    @functools.partial(
        pl.kernel,
        mesh=mesh,
        compiler_params=pltpu.CompilerParams(use_tc_tiling_on_sc=False),
        out_type=jax.ShapeDtypeStruct((NC, NPAD, W), jnp.float32),
        scratch_types=[
            pltpu.VMEM((NG, C), jnp.int32),
            pltpu.VMEM((NG, C), jnp.int32),
            pltpu.VMEM((C, W), jnp.float32),
            pltpu.VMEM((ZB, W), jnp.float32),
            pltpu.VMEM_SHARED((NPAD, W), jnp.float32),
            pltpu.SemaphoreType.DMA,
        ],
    )
    def sc_agg(h_hbm, src_hbm, dst_hbm, out_hbm,
               src_v, dst_v, rows_v, zbuf, agg_sh, sem):
        cid = lax.axis_index("c")
        sid = lax.axis_index("s")
        wid = sid * NC + cid

        def zrow(i, carry):
            for kk in range(W // 16):
                zbuf[i, pl.ds(kk * 16, 16)] = jnp.zeros((16,), jnp.float32)
            return carry
        lax.fori_loop(0, ZB, zrow, 0)

        def zcopy(j, carry):
            pltpu.sync_copy(zbuf, agg_sh.at[pl.ds(sid * RPT + j * ZB, ZB)])
            return carry
        lax.fori_loop(0, RPT // ZB, zcopy, 0)
        plsc.subcore_barrier()

        def group(g, carry):
            base = wid * NCHUNK + g * NG
            pltpu.sync_copy(src_hbm.at[pl.ds(base, NG)], src_v)
            pltpu.sync_copy(dst_hbm.at[pl.ds(base, NG)], dst_v)

            def chunk(j, c):
                pltpu.async_copy(h_hbm.at[src_v.at[j]], rows_v, sem).wait()
                pltpu.sync_copy(rows_v, agg_sh.at[dst_v.at[j]], add=True)
                return c
            lax.fori_loop(0, NG, chunk, 0)
            return carry
        lax.fori_loop(0, NGRP, group, 0)

        plsc.subcore_barrier()
        for j in range(RPT // ZR):
            r0 = sid * RPT + j * ZR
            pltpu.sync_copy(agg_sh.at[pl.ds(r0, ZR)],
                            out_hbm.at[cid, pl.ds(r0, ZR)])

    return sc_agg


_R = 1000


def _tc_layer1(parts, h, Wl, Wr, b):
    W1 = D + 16

    def body(p_ref, h_ref, wl_ref, wr_ref, b_ref, out_ref, invd_ref):
        p = p_ref[0] + p_ref[1]
        agg = p[:, :D]
        deg = p[:, D:D + 1]
        invd = 1.0 / jnp.maximum(deg, 1.0)
        mean = agg * invd
        y = (jnp.dot(mean, wl_ref[...], preferred_element_type=jnp.float32)
             + jnp.dot(h_ref[...], wr_ref[...],
                       preferred_element_type=jnp.float32)
             + b_ref[...])
        out_ref[...] = jnp.maximum(y, 0.0)
        invd_ref[...] = jnp.broadcast_to(invd, (_R, D))

    return pl.pallas_call(
        body,
        grid=(N // _R,),
        in_specs=[
            pl.BlockSpec((NC, _R, W1), lambda i: (0, i, 0)),
            pl.BlockSpec((_R, D), lambda i: (i, 0)),
            pl.BlockSpec((D, D), lambda i: (0, 0)),
            pl.BlockSpec((D, D), lambda i: (0, 0)),
            pl.BlockSpec((1, D), lambda i: (0, 0)),
        ],
        out_specs=[
            pl.BlockSpec((_R, D), lambda i: (i, 0)),
            pl.BlockSpec((_R, D), lambda i: (i, 0)),
        ],
        out_shape=[
            jax.ShapeDtypeStruct((N, D), jnp.float32),
            jax.ShapeDtypeStruct((N, D), jnp.float32),
        ],
    )(parts, h, Wl, Wr, b.reshape(1, D))


def _tc_layer(parts, h, invd, Wl, Wr, b):
    def body(p_ref, h_ref, invd_ref, wl_ref, wr_ref, b_ref, out_ref):
        mean = (p_ref[0] + p_ref[1]) * invd_ref[...]
        y = (jnp.dot(mean, wl_ref[...], preferred_element_type=jnp.float32)
             + jnp.dot(h_ref[...], wr_ref[...],
                       preferred_element_type=jnp.float32)
             + b_ref[...])
        out_ref[...] = jnp.maximum(y, 0.0)

    return pl.pallas_call(
        body,
        grid=(N // _R,),
        in_specs=[
            pl.BlockSpec((NC, _R, D), lambda i: (0, i, 0)),
            pl.BlockSpec((_R, D), lambda i: (i, 0)),
            pl.BlockSpec((_R, D), lambda i: (i, 0)),
            pl.BlockSpec((D, D), lambda i: (0, 0)),
            pl.BlockSpec((D, D), lambda i: (0, 0)),
            pl.BlockSpec((1, D), lambda i: (0, 0)),
        ],
        out_specs=pl.BlockSpec((_R, D), lambda i: (i, 0)),
        out_shape=jax.ShapeDtypeStruct((N, D), jnp.float32),
    )(parts, h, invd, Wl, Wr, b.reshape(1, D))


def kernel(x, edge_index, Wl1, Wr1, b1, Wl2, Wr2, b2, Wl3, Wr3, b3):
    src = edge_index[0].reshape(E // C, C)
    dst = edge_index[1].reshape(E // C, C)

    ones_col = jnp.ones((N, 1), jnp.float32)
    pad = jnp.zeros((N, 15), jnp.float32)
    x_aug = jnp.concatenate([x, ones_col, pad], axis=1)

    parts1 = _make_sc_agg(D + 16)(x_aug, src, dst)
    h1, invd = _tc_layer1(parts1, x, Wl1, Wr1, b1)

    sc_agg = _make_sc_agg(D)
    parts2 = sc_agg(h1, src, dst)
    h2 = _tc_layer(parts2, h1, invd, Wl2, Wr2, b2)

    parts3 = sc_agg(h2, src, dst)
    h3 = _tc_layer(parts3, h2, invd, Wl3, Wr3, b3)
    return h3
